# Initial kernel scaffold; baseline (speedup 1.0000x reference)
import jax
import jax.numpy as jnp
from jax.experimental import pallas as pl


def kernel(x, edge_index, batch, batch_size, params):
    raise NotImplementedError("write your pallas kernel here")


# speedup vs baseline: 8.1145x; 8.1145x over previous
import functools
import jax
import jax.numpy as jnp
from jax import lax
from jax.experimental import pallas as pl
from jax.experimental.pallas import tpu as pltpu
from jax.experimental.pallas import tpu_sc as plsc

N = 4096
E = 65536
H = 128
M0 = 64
M1 = 8
WD = 128

NC, NS, L = 2, 16, 16
NW = NC * NS
EPW = E // NW
CH = 128
NCH = EPW // CH
RPT = N // NS

_MESH = plsc.VectorSubcoreMesh(core_axis_name="c", subcore_axis_name="s")


def _wid(cid, sid):
    return sid * NC + cid


def _zero_rows(zbuf, shared, sid, n_rows, width):
    def zrow(r, _):
        for k in range(width // L):
            zbuf[r, pl.ds(k * L, L)] = jnp.zeros((L,), jnp.float32)
        return 0
    lax.fori_loop(0, 16, zrow, 0)

    def cp(k, _):
        pltpu.sync_copy(zbuf, shared.at[pl.ds(sid * n_rows + k * 16, 16)])
        return 0
    lax.fori_loop(0, n_rows // 16, cp, 0)


def _load_2d(src_1d_ref, dst2d, base):
    def cp(j, _):
        pltpu.sync_copy(src_1d_ref.at[pl.ds(base + j * CH, CH)], dst2d.at[j])
        return 0
    lax.fori_loop(0, NCH, cp, 0)


def _sc_edge_prep(edge, eidx):
    def body(edge_ref, eidx_ref, p_ref, degpart_ref,
             row2d, col2d, key2d, eid2d, onehot, zbuf, degacc):
        cid = lax.axis_index("c")
        sid = lax.axis_index("s")
        base = _wid(cid, sid) * EPW

        _zero_rows(zbuf, degacc, sid, RPT, WD)
        def orow(r, _):
            for k in range(WD // L):
                onehot[r, pl.ds(k * L, L)] = jnp.where(
                    lax.iota(jnp.int32, L) == (0 if k == 0 else -1), 1.0, 0.0)
            return 0
        lax.fori_loop(0, CH, orow, 0)
        plsc.subcore_barrier()

        _load_2d(edge_ref.at[0], row2d, base)
        _load_2d(edge_ref.at[1], col2d, base)
        _load_2d(eidx_ref, eid2d, base)

        def chunk(j, _):
            for g in range(CH // L):
                rv = row2d[j, pl.ds(g * L, L)]
                cv = col2d[j, pl.ds(g * L, L)]
                key2d[j, pl.ds(g * L, L)] = rv * N + cv
            pltpu.sync_copy(eid2d.at[j], p_ref.at[key2d.at[j]])
            pltpu.sync_copy(onehot, degacc.at[col2d.at[j]], add=True)
            return 0
        lax.fori_loop(0, NCH, chunk, 0)

        plsc.subcore_barrier()
        pltpu.sync_copy(degacc.at[pl.ds(sid * RPT, RPT)],
                        degpart_ref.at[cid, pl.ds(sid * RPT, RPT)])

    return pl.kernel(
        body,
        out_type=(
            jax.ShapeDtypeStruct((N * N,), jnp.int32),
            jax.ShapeDtypeStruct((NC, N, WD), jnp.float32),
        ),
        mesh=_MESH,
        scratch_types=[
            pltpu.VMEM((NCH, CH), jnp.int32),
            pltpu.VMEM((NCH, CH), jnp.int32),
            pltpu.VMEM((NCH, CH), jnp.int32),
            pltpu.VMEM((NCH, CH), jnp.int32),
            pltpu.VMEM((CH, WD), jnp.float32),
            pltpu.VMEM((16, WD), jnp.float32),
            pltpu.VMEM_SHARED((N, WD), jnp.float32),
        ],
    )(edge, eidx)


def _sc_mask_agg(p, edge, eidx, y):
    def body(p_ref, edge_ref, eidx_ref, y_ref, tgt_ref, maskpart_ref, acc_ref,
             row2d, col2d, key2d, eid2d, tgt2d, wv, onehot, rows_v, zbuf,
             maskacc, acc):
        cid = lax.axis_index("c")
        sid = lax.axis_index("s")
        base = _wid(cid, sid) * EPW

        _zero_rows(zbuf, maskacc.at[pl.ds(0, N)], sid, RPT, WD)
        @pl.when(sid == 0)
        def _():
            pltpu.sync_copy(zbuf, maskacc.at[pl.ds(N, 16)])
        _zero_rows(zbuf, acc, sid, RPT, WD)
        def orow(r, _):
            for k in range(WD // L):
                onehot[r, pl.ds(k * L, L)] = jnp.where(
                    lax.iota(jnp.int32, L) == (0 if k == 0 else -1), 1.0, 0.0)
            return 0
        lax.fori_loop(0, CH, orow, 0)
        plsc.subcore_barrier()

        _load_2d(edge_ref.at[0], row2d, base)
        _load_2d(edge_ref.at[1], col2d, base)
        _load_2d(eidx_ref, eid2d, base)

        def chunk(j, _):
            for g in range(CH // L):
                rv = row2d[j, pl.ds(g * L, L)]
                cv = col2d[j, pl.ds(g * L, L)]
                key2d[j, pl.ds(g * L, L)] = rv * N + cv
            pltpu.sync_copy(p_ref.at[key2d.at[j]], wv)
            for g in range(CH // L):
                win = wv[pl.ds(g * L, L)]
                ev = eid2d[j, pl.ds(g * L, L)]
                rv = row2d[j, pl.ds(g * L, L)]
                m = win == ev
                tgt2d[j, pl.ds(g * L, L)] = jnp.where(m, rv, N)
            pltpu.sync_copy(onehot, maskacc.at[tgt2d.at[j]], add=True)
            pltpu.sync_copy(y_ref.at[row2d.at[j]], rows_v)
            pltpu.sync_copy(rows_v, acc.at[col2d.at[j]], add=True)
            pltpu.sync_copy(tgt2d.at[j], tgt_ref.at[pl.ds(base + j * CH, CH)])
            return 0
        lax.fori_loop(0, NCH, chunk, 0)

        plsc.subcore_barrier()
        pltpu.sync_copy(maskacc.at[pl.ds(sid * RPT, RPT)],
                        maskpart_ref.at[cid, pl.ds(sid * RPT, RPT)])
        pltpu.sync_copy(acc.at[pl.ds(sid * RPT, RPT)],
                        acc_ref.at[cid, pl.ds(sid * RPT, RPT)])

    return pl.kernel(
        body,
        out_type=(
            jax.ShapeDtypeStruct((E,), jnp.int32),
            jax.ShapeDtypeStruct((NC, N, WD), jnp.float32),
            jax.ShapeDtypeStruct((NC, N, H), jnp.float32),
        ),
        mesh=_MESH,
        scratch_types=[
            pltpu.VMEM((NCH, CH), jnp.int32),
            pltpu.VMEM((NCH, CH), jnp.int32),
            pltpu.VMEM((NCH, CH), jnp.int32),
            pltpu.VMEM((NCH, CH), jnp.int32),
            pltpu.VMEM((NCH, CH), jnp.int32),
            pltpu.VMEM((CH,), jnp.int32),
            pltpu.VMEM((CH, WD), jnp.float32),
            pltpu.VMEM((CH, H), jnp.float32),
            pltpu.VMEM((16, WD), jnp.float32),
            pltpu.VMEM_SHARED((N + 16, WD), jnp.float32),
            pltpu.VMEM_SHARED((N, H), jnp.float32),
        ],
    )(p, edge, eidx, y)


def _sc_conv_agg(edge, y):
    def body(edge_ref, y_ref, acc_ref, row2d, col2d, rows_v, zbuf, acc):
        cid = lax.axis_index("c")
        sid = lax.axis_index("s")
        base = _wid(cid, sid) * EPW

        _zero_rows(zbuf, acc, sid, RPT, WD)
        plsc.subcore_barrier()

        _load_2d(edge_ref.at[0], row2d, base)
        _load_2d(edge_ref.at[1], col2d, base)

        def chunk(j, _):
            pltpu.sync_copy(y_ref.at[row2d.at[j]], rows_v)
            pltpu.sync_copy(rows_v, acc.at[col2d.at[j]], add=True)
            return 0
        lax.fori_loop(0, NCH, chunk, 0)

        plsc.subcore_barrier()
        pltpu.sync_copy(acc.at[pl.ds(sid * RPT, RPT)],
                        acc_ref.at[cid, pl.ds(sid * RPT, RPT)])

    return pl.kernel(
        body,
        out_type=jax.ShapeDtypeStruct((NC, N, H), jnp.float32),
        mesh=_MESH,
        scratch_types=[
            pltpu.VMEM((NCH, CH), jnp.int32),
            pltpu.VMEM((NCH, CH), jnp.int32),
            pltpu.VMEM((CH, H), jnp.float32),
            pltpu.VMEM((16, WD), jnp.float32),
            pltpu.VMEM_SHARED((N, H), jnp.float32),
        ],
    )(edge, y)


def _sc_pool_agg(edge, tgt, sd80):
    ACC_R = N + 16

    def body(edge_ref, tgt_ref, sd_ref, tpart_ref, col2d, tgt2d, rows_v, zbuf,
             acc):
        cid = lax.axis_index("c")
        sid = lax.axis_index("s")
        base = _wid(cid, sid) * EPW

        _zero_rows(zbuf, acc.at[pl.ds(0, N)], sid, RPT, WD)
        @pl.when(sid == 0)
        def _():
            pltpu.sync_copy(zbuf, acc.at[pl.ds(N, 16)])
        plsc.subcore_barrier()

        _load_2d(edge_ref.at[1], col2d, base)
        _load_2d(tgt_ref, tgt2d, base)

        def chunk(j, _):
            pltpu.sync_copy(sd_ref.at[col2d.at[j]], rows_v)
            pltpu.sync_copy(rows_v, acc.at[tgt2d.at[j]], add=True)
            return 0
        lax.fori_loop(0, NCH, chunk, 0)

        plsc.subcore_barrier()
        pltpu.sync_copy(acc.at[pl.ds(sid * RPT, RPT)],
                        tpart_ref.at[cid, pl.ds(sid * RPT, RPT)])

    return pl.kernel(
        body,
        out_type=jax.ShapeDtypeStruct((NC, N, WD), jnp.float32),
        mesh=_MESH,
        scratch_types=[
            pltpu.VMEM((NCH, CH), jnp.int32),
            pltpu.VMEM((NCH, CH), jnp.int32),
            pltpu.VMEM((CH, WD), jnp.float32),
            pltpu.VMEM((16, WD), jnp.float32),
            pltpu.VMEM_SHARED((ACC_R, WD), jnp.float32),
        ],
    )(edge, tgt, sd80)


_BLK = 512
_GRID = N // _BLK


def _ln(x, g, b, eps=1e-5):
    mu = jnp.mean(x, axis=-1, keepdims=True)
    var = jnp.mean((x - mu) ** 2, axis=-1, keepdims=True)
    return (x - mu) * lax.rsqrt(var + eps) * g + b


def _dis_of(degpart_blk):
    deg = jnp.sum(degpart_blk[0], axis=-1, keepdims=True) + \
        jnp.sum(degpart_blk[1], axis=-1, keepdims=True) + 1.0
    return lax.rsqrt(deg)


def _tc_input(x, degpart, in_W, in_b, ln_g, ln_b, c1_W):
    def body(x_ref, dp_ref, w_ref, b_ref, g_ref, lb_ref, w1_ref, y_ref):
        xb = x_ref[...]
        acc = jnp.zeros((_BLK, H), jnp.float32)
        for j in range(4):
            acc = acc + xb[:, j:j + 1] * w_ref[j:j + 1, :]
        h0 = jnp.tanh(_ln(acc + b_ref[...].reshape(1, H),
                          g_ref[...].reshape(1, H), lb_ref[...].reshape(1, H)))
        dis = _dis_of(dp_ref[...])
        y_ref[...] = dis * jnp.dot(h0, w1_ref[...],
                                   preferred_element_type=jnp.float32)

    return pl.pallas_call(
        body,
        grid=(_GRID,),
        in_specs=[
            pl.BlockSpec((_BLK, 4), lambda i: (i, 0)),
            pl.BlockSpec((NC, _BLK, WD), lambda i: (0, i, 0)),
            pl.BlockSpec((4, H), lambda i: (0, 0)),
            pl.BlockSpec((H,), lambda i: (0,)),
            pl.BlockSpec((H,), lambda i: (0,)),
            pl.BlockSpec((H,), lambda i: (0,)),
            pl.BlockSpec((H, H), lambda i: (0, 0)),
        ],
        out_specs=pl.BlockSpec((_BLK, H), lambda i: (i, 0)),
        out_shape=jax.ShapeDtypeStruct((N, H), jnp.float32),
    )(x, degpart, in_W, in_b, ln_g, ln_b, c1_W)


def _tc_conv_step(accpair, yprev, degpart, b_prev, W_next):
    def body(a_ref, y_ref, dp_ref, b_ref, w_ref, o_ref):
        dis = _dis_of(dp_ref[...])
        h = jax.nn.relu(dis * (a_ref[0] + a_ref[1] + y_ref[...]) +
                        b_ref[...].reshape(1, H))
        o_ref[...] = dis * jnp.dot(h, w_ref[...],
                                   preferred_element_type=jnp.float32)

    return pl.pallas_call(
        body,
        grid=(_GRID,),
        in_specs=[
            pl.BlockSpec((NC, _BLK, H), lambda i: (0, i, 0)),
            pl.BlockSpec((_BLK, H), lambda i: (i, 0)),
            pl.BlockSpec((NC, _BLK, WD), lambda i: (0, i, 0)),
            pl.BlockSpec((H,), lambda i: (0,)),
            pl.BlockSpec((H, H), lambda i: (0, 0)),
        ],
        out_specs=pl.BlockSpec((_BLK, H), lambda i: (i, 0)),
        out_shape=jax.ShapeDtypeStruct((N, H), jnp.float32),
    )(accpair, yprev, degpart, b_prev, W_next)


def _tc_assign(accpair, yprev, degpart, maskpart, b3,
               a1_W1, a1_b1, ln1_g, ln1_b, a1_W2, a1_b2, ln2_g, ln2_b):
    def body(a_ref, y_ref, dp_ref, mp_ref, b3_ref, w1_ref, b1_ref, g1_ref,
             lb1_ref, w2_ref, b2_ref, g2_ref, lb2_ref,
             h_ref, s_ref, sd_ref):
        dis = _dis_of(dp_ref[...])
        h3 = dis * (a_ref[0] + a_ref[1] + y_ref[...]) + b3_ref[...].reshape(1, H)
        h_ref[...] = h3
        t = jnp.tanh(_ln(jnp.dot(h3, w1_ref[...],
                                 preferred_element_type=jnp.float32) +
                         b1_ref[...].reshape(1, H),
                         g1_ref[...].reshape(1, H), lb1_ref[...].reshape(1, H)))
        s = jax.nn.relu(_ln(jnp.dot(t, w2_ref[...],
                                    preferred_element_type=jnp.float32) +
                            b2_ref[...].reshape(1, M0),
                            g2_ref[...].reshape(1, M0),
                            lb2_ref[...].reshape(1, M0)))
        s = s - jnp.max(s, axis=-1, keepdims=True)
        es = jnp.exp(s)
        S = es / jnp.sum(es, axis=-1, keepdims=True)
        s_ref[...] = S
        degD = jnp.sum(mp_ref[0], axis=-1, keepdims=True) + \
            jnp.sum(mp_ref[1], axis=-1, keepdims=True) + 1.0
        disD = lax.rsqrt(degD)
        sd_ref[:, 0:M0] = disD * S
        sd_ref[:, M0:M0 + 1] = disD
        sd_ref[:, M0 + 1:WD] = jnp.zeros((_BLK, WD - M0 - 1), jnp.float32)

    return pl.pallas_call(
        body,
        grid=(_GRID,),
        in_specs=[
            pl.BlockSpec((NC, _BLK, H), lambda i: (0, i, 0)),
            pl.BlockSpec((_BLK, H), lambda i: (i, 0)),
            pl.BlockSpec((NC, _BLK, WD), lambda i: (0, i, 0)),
            pl.BlockSpec((NC, _BLK, WD), lambda i: (0, i, 0)),
            pl.BlockSpec((H,), lambda i: (0,)),
            pl.BlockSpec((H, H), lambda i: (0, 0)),
            pl.BlockSpec((H,), lambda i: (0,)),
            pl.BlockSpec((H,), lambda i: (0,)),
            pl.BlockSpec((H,), lambda i: (0,)),
            pl.BlockSpec((H, M0), lambda i: (0, 0)),
            pl.BlockSpec((M0,), lambda i: (0,)),
            pl.BlockSpec((M0,), lambda i: (0,)),
            pl.BlockSpec((M0,), lambda i: (0,)),
        ],
        out_specs=[
            pl.BlockSpec((_BLK, H), lambda i: (i, 0)),
            pl.BlockSpec((_BLK, M0), lambda i: (i, 0)),
            pl.BlockSpec((_BLK, WD), lambda i: (i, 0)),
        ],
        out_shape=[
            jax.ShapeDtypeStruct((N, H), jnp.float32),
            jax.ShapeDtypeStruct((N, M0), jnp.float32),
            jax.ShapeDtypeStruct((N, WD), jnp.float32),
        ],
    )(accpair, yprev, degpart, maskpart, b3,
      a1_W1, a1_b1, ln1_g, ln1_b, a1_W2, a1_b2, ln2_g, ln2_b)


def _tc_tail(h3, S, sd80, tpart, p):
    def body(h_ref, s_ref, sd_ref, t_ref,
             c4w, c4b, c5w, c5b, c6w, c6b,
             a2w1, a2b1, a2g1, a2lb1, a2w2, a2b2, a2g2, a2lb2,
             ow1, ob1, og1, olb1, ow2, ob2, og2, olb2, ow3, ob3,
             gp_ref, ml1_ref, ol1_ref, ml2_ref, ol2_ref):
        Sm = s_ref[...]
        SD = sd_ref[:, 0:M0]
        disD = sd_ref[:, M0:M0 + 1]
        Tm = t_ref[0, :, 0:M0] + t_ref[1, :, 0:M0]
        rm = t_ref[0, :, M0:M0 + 1] + t_ref[1, :, M0:M0 + 1]

        cdim = (((0,), (0,)), ((), ()))
        out_adj = lax.dot_general(SD, Tm + SD, cdim,
                                  preferred_element_type=jnp.float32)
        out0 = lax.dot_general(Sm, h_ref[...], cdim,
                               preferred_element_type=jnp.float32)
        ss = lax.dot_general(Sm, Sm, cdim,
                             preferred_element_type=jnp.float32)
        d_flat = disD * (disD + rm)

        i0 = lax.broadcasted_iota(jnp.int32, (M0, M0), 0)
        i1 = lax.broadcasted_iota(jnp.int32, (M0, M0), 1)
        eye0 = jnp.where(i0 == i1, 1.0, 0.0)

        mincut_num = jnp.sum(out_adj * eye0)
        mincut_den = jnp.sum(d_flat * jnp.sum(Sm * Sm, axis=-1, keepdims=True))
        ml1 = -(mincut_num / mincut_den)

        ss_norm = jnp.sqrt(jnp.sum(ss * ss))
        ol1 = jnp.sqrt(jnp.sum((ss / ss_norm - eye0 / jnp.sqrt(
            jnp.float32(M0))) ** 2))

        out_adj = out_adj * (1.0 - eye0)
        dcol = jnp.sqrt(jnp.sum(out_adj, axis=-1, keepdims=True)) + 1e-15
        out_adj = out_adj / dcol / dcol.reshape(1, M0)

        def dense_gcn(xz, adj, W, b):
            adj = adj * (1.0 - eye0) + eye0
            dd = jnp.sum(adj, axis=-1, keepdims=True)
            ddis = jnp.where(dd > 0, lax.rsqrt(dd), 0.0)
            adjn = ddis * adj * ddis.reshape(1, M0)
            return jnp.dot(adjn, jnp.dot(xz, W,
                                         preferred_element_type=jnp.float32),
                           preferred_element_type=jnp.float32) + b.reshape(1, -1)

        z = jax.nn.relu(dense_gcn(out0, out_adj, c4w[...], c4b[...]))
        z = jax.nn.relu(dense_gcn(z, out_adj, c5w[...], c5b[...]))
        z = dense_gcn(z, out_adj, c6w[...], c6b[...])

        t2 = jnp.tanh(_ln(jnp.dot(z, a2w1[...],
                                  preferred_element_type=jnp.float32) +
                          a2b1[...].reshape(1, H),
                          a2g1[...].reshape(1, H), a2lb1[...].reshape(1, H)))
        s2 = jax.nn.relu(_ln(jnp.dot(t2, a2w2[...],
                                     preferred_element_type=jnp.float32) +
                             a2b2[...].reshape(1, M1),
                             a2g2[...].reshape(1, M1), a2lb2[...].reshape(1, M1)))
        s2 = s2 - jnp.max(s2, axis=-1, keepdims=True)
        e2 = jnp.exp(s2)
        S2 = e2 / jnp.sum(e2, axis=-1, keepdims=True)

        out2 = lax.dot_general(S2, z, cdim, preferred_element_type=jnp.float32)
        oa2 = lax.dot_general(S2, jnp.dot(out_adj, S2,
                                          preferred_element_type=jnp.float32),
                              cdim, preferred_element_type=jnp.float32)
        j0 = lax.broadcasted_iota(jnp.int32, (M1, M1), 0)
        j1 = lax.broadcasted_iota(jnp.int32, (M1, M1), 1)
        eye1 = jnp.where(j0 == j1, 1.0, 0.0)
        mn2 = jnp.sum(oa2 * eye1)
        dfl2 = jnp.sum(out_adj, axis=-1, keepdims=True)
        md2 = jnp.sum(dfl2 * jnp.sum(S2 * S2, axis=-1, keepdims=True))
        ml2 = -(mn2 / md2)
        ss2 = lax.dot_general(S2, S2, cdim, preferred_element_type=jnp.float32)
        ss2n = jnp.sqrt(jnp.sum(ss2 * ss2))
        ol2 = jnp.sqrt(jnp.sum((ss2 / ss2n - eye1 / jnp.sqrt(
            jnp.float32(M1))) ** 2))

        pooled = jnp.mean(out2, axis=0, keepdims=True)
        g1 = jnp.tanh(_ln(jnp.dot(pooled, ow1[...],
                                  preferred_element_type=jnp.float32) +
                          ob1[...].reshape(1, H),
                          og1[...].reshape(1, H), olb1[...].reshape(1, H)))
        g2 = jnp.tanh(_ln(jnp.dot(g1, ow2[...],
                                  preferred_element_type=jnp.float32) +
                          ob2[...].reshape(1, H),
                          og2[...].reshape(1, H), olb2[...].reshape(1, H)))
        gp = jnp.dot(g2, ow3[...], preferred_element_type=jnp.float32) + \
            ob3[...].reshape(1, 1)

        gp_ref[...] = gp
        ml1_ref[...] = jnp.reshape(ml1, (1, 1))
        ol1_ref[...] = jnp.reshape(ol1, (1, 1))
        ml2_ref[...] = jnp.reshape(ml2, (1, 1))
        ol2_ref[...] = jnp.reshape(ol2, (1, 1))

    args = (h3, S, sd80, tpart,
            p['c4_W'], p['c4_b'], p['c5_W'], p['c5_b'], p['c6_W'], p['c6_b'],
            p['a2_W1'], p['a2_b1'], p['a2_ln1_g'], p['a2_ln1_b'],
            p['a2_W2'], p['a2_b2'], p['a2_ln2_g'], p['a2_ln2_b'],
            p['o_W1'], p['o_b1'], p['o_ln1_g'], p['o_ln1_b'],
            p['o_W2'], p['o_b2'], p['o_ln2_g'], p['o_ln2_b'],
            p['o_W3'], p['o_b3'])
    out_shape = [jax.ShapeDtypeStruct((1, 1), jnp.float32)] * 5
    return pl.pallas_call(body, out_shape=out_shape)(*args)


def kernel(x, edge_index, batch, batch_size, params):
    p = params
    edge = edge_index.astype(jnp.int32)
    eidx = lax.iota(jnp.int32, E)

    pwin, degpart = _sc_edge_prep(edge, eidx)
    y1p = _tc_input(x, degpart, p['in_W'], p['in_b'], p['in_ln_g'],
                    p['in_ln_b'], p['c1_W'])
    tgt, maskpart, acc1 = _sc_mask_agg(pwin, edge, eidx, y1p)
    y2p = _tc_conv_step(acc1, y1p, degpart, p['c1_b'], p['c2_W'])
    acc2 = _sc_conv_agg(edge, y2p)
    y3p = _tc_conv_step(acc2, y2p, degpart, p['c2_b'], p['c3_W'])
    acc3 = _sc_conv_agg(edge, y3p)
    h3, S, sd80 = _tc_assign(acc3, y3p, degpart, maskpart, p['c3_b'],
                             p['a1_W1'], p['a1_b1'], p['a1_ln1_g'],
                             p['a1_ln1_b'], p['a1_W2'], p['a1_b2'],
                             p['a1_ln2_g'], p['a1_ln2_b'])
    tpart = _sc_pool_agg(edge, tgt, sd80)
    gp, ml1, ol1, ml2, ol2 = _tc_tail(h3, S, sd80, tpart, p)
    return (gp, ml1.reshape(()), ol1.reshape(()), ml2.reshape(()),
            ol2.reshape(()))


# speedup vs baseline: 12.6239x; 1.5557x over previous
import functools
import jax
import jax.numpy as jnp
from jax import lax
from jax.experimental import pallas as pl
from jax.experimental.pallas import tpu as pltpu
from jax.experimental.pallas import tpu_sc as plsc

N = 4096
E = 65536
H = 128
M0 = 64
M1 = 8
WD = 128

NC, NS, L = 2, 16, 16
NW = NC * NS
EPW = E // NW
CH = 128
NCH = EPW // CH
RPT = N // NS

_MESH = plsc.VectorSubcoreMesh(core_axis_name="c", subcore_axis_name="s")


def _wid(cid, sid):
    return sid * NC + cid


def _zero_rows(zbuf, shared, sid, n_rows, sem):
    def zrow(r, _):
        for k in range(WD // L):
            zbuf[r, pl.ds(k * L, L)] = jnp.zeros((L,), jnp.float32)
        return 0
    lax.fori_loop(0, 16, zrow, 0)

    def fire(k, _):
        pltpu.async_copy(zbuf, shared.at[pl.ds(sid * n_rows + k * 16, 16)], sem)
        return 0
    lax.fori_loop(0, n_rows // 16, fire, 0)

    def drain(k, _):
        pltpu.make_async_copy(zbuf, shared.at[pl.ds(sid * n_rows, 16)],
                              sem).wait()
        return 0
    lax.fori_loop(0, n_rows // 16, drain, 0)


def _load_2d(src_1d_ref, dst2d, base, sem):
    def fire(j, _):
        pltpu.async_copy(src_1d_ref.at[pl.ds(base + j * CH, CH)], dst2d.at[j],
                         sem)
        return 0
    lax.fori_loop(0, NCH, fire, 0)

    def drain(j, _):
        pltpu.make_async_copy(src_1d_ref.at[pl.ds(base, CH)], dst2d.at[0],
                              sem).wait()
        return 0
    lax.fori_loop(0, NCH, drain, 0)


def _pipe_agg(table_ref, gidx2d, sidx2d, acc, bufs, sg, ss):
    nb = len(bufs)
    for b in range(nb):
        pltpu.async_copy(table_ref.at[gidx2d.at[b]], bufs[b], sg)

    def step(i, _):
        j0 = i * nb
        for b in range(nb):
            jj = j0 + b
            pltpu.make_async_copy(table_ref.at[gidx2d.at[jj]], bufs[b],
                                  sg).wait()
            pltpu.async_copy(bufs[b], acc.at[sidx2d.at[jj]], ss, add=True)

            @pl.when(jj + nb < NCH)
            def _():
                pltpu.make_async_copy(bufs[b], acc.at[sidx2d.at[jj]],
                                      ss).wait()
                pltpu.async_copy(table_ref.at[gidx2d.at[jj + nb]], bufs[b], sg)
        return 0
    lax.fori_loop(0, NCH // nb, step, 0)
    for b in range(nb):
        pltpu.make_async_copy(bufs[b], acc.at[sidx2d.at[0]], ss).wait()


def _sc_edge_prep(edge, eidx):
    def body(edge_ref, eidx_ref, p_ref, degpart_ref,
             row2d, col2d, key2d, eid2d, onehot, zbuf, degacc,
             sem_l, sem_p, sem_h, sem_z):
        cid = lax.axis_index("c")
        sid = lax.axis_index("s")
        base = _wid(cid, sid) * EPW

        _zero_rows(zbuf, degacc, sid, RPT, sem_z)
        def orow(r, _):
            for k in range(WD // L):
                onehot[r, pl.ds(k * L, L)] = jnp.where(
                    lax.iota(jnp.int32, L) == (0 if k == 0 else -1), 1.0, 0.0)
            return 0
        lax.fori_loop(0, CH, orow, 0)
        plsc.subcore_barrier()

        _load_2d(edge_ref.at[0], row2d, base, sem_l)
        _load_2d(edge_ref.at[1], col2d, base, sem_l)
        _load_2d(eidx_ref, eid2d, base, sem_l)

        def chunk(j, _):
            for g in range(CH // L):
                rv = row2d[j, pl.ds(g * L, L)]
                cv = col2d[j, pl.ds(g * L, L)]
                key2d[j, pl.ds(g * L, L)] = rv * N + cv
            pltpu.async_copy(eid2d.at[j], p_ref.at[key2d.at[j]], sem_p)
            pltpu.async_copy(onehot, degacc.at[col2d.at[j]], sem_h, add=True)
            return 0
        lax.fori_loop(0, NCH, chunk, 0)

        def drain(j, _):
            pltpu.make_async_copy(eid2d.at[0], p_ref.at[key2d.at[0]],
                                  sem_p).wait()
            pltpu.make_async_copy(onehot, degacc.at[col2d.at[0]],
                                  sem_h).wait()
            return 0
        lax.fori_loop(0, NCH, drain, 0)

        plsc.subcore_barrier()
        pltpu.sync_copy(degacc.at[pl.ds(sid * RPT, RPT)],
                        degpart_ref.at[cid, pl.ds(sid * RPT, RPT)])

    return pl.kernel(
        body,
        out_type=(
            jax.ShapeDtypeStruct((N * N,), jnp.int32),
            jax.ShapeDtypeStruct((NC, N, WD), jnp.float32),
        ),
        mesh=_MESH,
        scratch_types=[
            pltpu.VMEM((NCH, CH), jnp.int32),
            pltpu.VMEM((NCH, CH), jnp.int32),
            pltpu.VMEM((NCH, CH), jnp.int32),
            pltpu.VMEM((NCH, CH), jnp.int32),
            pltpu.VMEM((CH, WD), jnp.float32),
            pltpu.VMEM((16, WD), jnp.float32),
            pltpu.VMEM_SHARED((N, WD), jnp.float32),
            pltpu.SemaphoreType.DMA,
            pltpu.SemaphoreType.DMA,
            pltpu.SemaphoreType.DMA,
            pltpu.SemaphoreType.DMA,
        ],
    )(edge, eidx)


def _sc_mask_agg(p, edge, eidx, y):
    def body(p_ref, edge_ref, eidx_ref, y_ref, tgt_ref, maskpart_ref, acc_ref,
             row2d, col2d, key2d, eid2d, tgt2d, wv0, wv1, onehot, rb0, rb1,
             zbuf, maskacc, acc, sem_l, sem_w, sem_h, sem_t, sg, ss, sem_z):
        cid = lax.axis_index("c")
        sid = lax.axis_index("s")
        base = _wid(cid, sid) * EPW

        _zero_rows(zbuf, maskacc.at[pl.ds(0, N)], sid, RPT, sem_z)
        @pl.when(sid == 0)
        def _():
            pltpu.sync_copy(zbuf, maskacc.at[pl.ds(N, 16)])
        _zero_rows(zbuf, acc, sid, RPT, sem_z)
        def orow(r, _):
            for k in range(WD // L):
                onehot[r, pl.ds(k * L, L)] = jnp.where(
                    lax.iota(jnp.int32, L) == (0 if k == 0 else -1), 1.0, 0.0)
            return 0
        lax.fori_loop(0, CH, orow, 0)
        plsc.subcore_barrier()

        _load_2d(edge_ref.at[0], row2d, base, sem_l)
        _load_2d(edge_ref.at[1], col2d, base, sem_l)
        _load_2d(eidx_ref, eid2d, base, sem_l)

        def keys(j, _):
            for g in range(CH // L):
                rv = row2d[j, pl.ds(g * L, L)]
                cv = col2d[j, pl.ds(g * L, L)]
                key2d[j, pl.ds(g * L, L)] = rv * N + cv
            return 0
        lax.fori_loop(0, NCH, keys, 0)

        wvs = (wv0, wv1)
        rbs = (rb0, rb1)
        for b in range(2):
            pltpu.async_copy(p_ref.at[key2d.at[b]], wvs[b], sem_w)
            pltpu.async_copy(y_ref.at[row2d.at[b]], rbs[b], sg)

        def chunk(i, _):
            j0 = i * 2
            for b in range(2):
                jj = j0 + b
                pltpu.make_async_copy(p_ref.at[key2d.at[jj]], wvs[b],
                                      sem_w).wait()
                for g in range(CH // L):
                    win = wvs[b][pl.ds(g * L, L)]
                    ev = eid2d[jj, pl.ds(g * L, L)]
                    rv = row2d[jj, pl.ds(g * L, L)]
                    m = win == ev
                    tgt2d[jj, pl.ds(g * L, L)] = jnp.where(m, rv, N)
                pltpu.async_copy(onehot, maskacc.at[tgt2d.at[jj]], sem_h,
                                 add=True)
                pltpu.async_copy(tgt2d.at[jj],
                                 tgt_ref.at[pl.ds(base + jj * CH, CH)], sem_t)
                pltpu.make_async_copy(y_ref.at[row2d.at[jj]], rbs[b],
                                      sg).wait()
                pltpu.async_copy(rbs[b], acc.at[col2d.at[jj]], ss, add=True)

                @pl.when(jj + 2 < NCH)
                def _():
                    pltpu.make_async_copy(rbs[b], acc.at[col2d.at[jj]],
                                          ss).wait()
                    pltpu.async_copy(p_ref.at[key2d.at[jj + 2]], wvs[b], sem_w)
                    pltpu.async_copy(y_ref.at[row2d.at[jj + 2]], rbs[b], sg)
            return 0
        lax.fori_loop(0, NCH // 2, chunk, 0)

        pltpu.make_async_copy(rb0, acc.at[col2d.at[0]], ss).wait()
        pltpu.make_async_copy(rb1, acc.at[col2d.at[0]], ss).wait()
        def drain(j, _):
            pltpu.make_async_copy(onehot, maskacc.at[tgt2d.at[0]],
                                  sem_h).wait()
            pltpu.make_async_copy(tgt2d.at[0],
                                  tgt_ref.at[pl.ds(base, CH)], sem_t).wait()
            return 0
        lax.fori_loop(0, NCH, drain, 0)

        plsc.subcore_barrier()
        pltpu.sync_copy(maskacc.at[pl.ds(sid * RPT, RPT)],
                        maskpart_ref.at[cid, pl.ds(sid * RPT, RPT)])
        pltpu.sync_copy(acc.at[pl.ds(sid * RPT, RPT)],
                        acc_ref.at[cid, pl.ds(sid * RPT, RPT)])

    return pl.kernel(
        body,
        out_type=(
            jax.ShapeDtypeStruct((E,), jnp.int32),
            jax.ShapeDtypeStruct((NC, N, WD), jnp.float32),
            jax.ShapeDtypeStruct((NC, N, H), jnp.float32),
        ),
        mesh=_MESH,
        scratch_types=[
            pltpu.VMEM((NCH, CH), jnp.int32),
            pltpu.VMEM((NCH, CH), jnp.int32),
            pltpu.VMEM((NCH, CH), jnp.int32),
            pltpu.VMEM((NCH, CH), jnp.int32),
            pltpu.VMEM((NCH, CH), jnp.int32),
            pltpu.VMEM((CH,), jnp.int32),
            pltpu.VMEM((CH,), jnp.int32),
            pltpu.VMEM((CH, WD), jnp.float32),
            pltpu.VMEM((CH, H), jnp.float32),
            pltpu.VMEM((CH, H), jnp.float32),
            pltpu.VMEM((16, WD), jnp.float32),
            pltpu.VMEM_SHARED((N + 16, WD), jnp.float32),
            pltpu.VMEM_SHARED((N, H), jnp.float32),
            pltpu.SemaphoreType.DMA,
            pltpu.SemaphoreType.DMA,
            pltpu.SemaphoreType.DMA,
            pltpu.SemaphoreType.DMA,
            pltpu.SemaphoreType.DMA,
            pltpu.SemaphoreType.DMA,
            pltpu.SemaphoreType.DMA,
        ],
    )(p, edge, eidx, y)


def _sc_conv_agg(edge, y):
    def body(edge_ref, y_ref, acc_ref, row2d, col2d, rb0, rb1, zbuf, acc,
             sem_l, sg, ss, sem_z):
        cid = lax.axis_index("c")
        sid = lax.axis_index("s")
        base = _wid(cid, sid) * EPW

        _zero_rows(zbuf, acc, sid, RPT, sem_z)
        plsc.subcore_barrier()

        _load_2d(edge_ref.at[0], row2d, base, sem_l)
        _load_2d(edge_ref.at[1], col2d, base, sem_l)

        _pipe_agg(y_ref, row2d, col2d, acc, (rb0, rb1), sg, ss)

        plsc.subcore_barrier()
        pltpu.sync_copy(acc.at[pl.ds(sid * RPT, RPT)],
                        acc_ref.at[cid, pl.ds(sid * RPT, RPT)])

    return pl.kernel(
        body,
        out_type=jax.ShapeDtypeStruct((NC, N, H), jnp.float32),
        mesh=_MESH,
        scratch_types=[
            pltpu.VMEM((NCH, CH), jnp.int32),
            pltpu.VMEM((NCH, CH), jnp.int32),
            pltpu.VMEM((CH, H), jnp.float32),
            pltpu.VMEM((CH, H), jnp.float32),
            pltpu.VMEM((16, WD), jnp.float32),
            pltpu.VMEM_SHARED((N, H), jnp.float32),
            pltpu.SemaphoreType.DMA,
            pltpu.SemaphoreType.DMA,
            pltpu.SemaphoreType.DMA,
            pltpu.SemaphoreType.DMA,
        ],
    )(edge, y)


def _sc_pool_agg(edge, tgt, sd80):
    ACC_R = N + 16

    def body(edge_ref, tgt_ref, sd_ref, tpart_ref, col2d, tgt2d, rb0, rb1,
             zbuf, acc, sem_l, sg, ss, sem_z):
        cid = lax.axis_index("c")
        sid = lax.axis_index("s")
        base = _wid(cid, sid) * EPW

        _zero_rows(zbuf, acc.at[pl.ds(0, N)], sid, RPT, sem_z)
        @pl.when(sid == 0)
        def _():
            pltpu.sync_copy(zbuf, acc.at[pl.ds(N, 16)])
        plsc.subcore_barrier()

        _load_2d(edge_ref.at[1], col2d, base, sem_l)
        _load_2d(tgt_ref, tgt2d, base, sem_l)

        _pipe_agg(sd_ref, col2d, tgt2d, acc, (rb0, rb1), sg, ss)

        plsc.subcore_barrier()
        pltpu.sync_copy(acc.at[pl.ds(sid * RPT, RPT)],
                        tpart_ref.at[cid, pl.ds(sid * RPT, RPT)])

    return pl.kernel(
        body,
        out_type=jax.ShapeDtypeStruct((NC, N, WD), jnp.float32),
        mesh=_MESH,
        scratch_types=[
            pltpu.VMEM((NCH, CH), jnp.int32),
            pltpu.VMEM((NCH, CH), jnp.int32),
            pltpu.VMEM((CH, WD), jnp.float32),
            pltpu.VMEM((CH, WD), jnp.float32),
            pltpu.VMEM((16, WD), jnp.float32),
            pltpu.VMEM_SHARED((ACC_R, WD), jnp.float32),
            pltpu.SemaphoreType.DMA,
            pltpu.SemaphoreType.DMA,
            pltpu.SemaphoreType.DMA,
            pltpu.SemaphoreType.DMA,
        ],
    )(edge, tgt, sd80)


_BLK = 512
_GRID = N // _BLK


def _ln(x, g, b, eps=1e-5):
    mu = jnp.mean(x, axis=-1, keepdims=True)
    var = jnp.mean((x - mu) ** 2, axis=-1, keepdims=True)
    return (x - mu) * lax.rsqrt(var + eps) * g + b


def _dis_of(degpart_blk):
    deg = jnp.sum(degpart_blk[0], axis=-1, keepdims=True) + \
        jnp.sum(degpart_blk[1], axis=-1, keepdims=True) + 1.0
    return lax.rsqrt(deg)


def _tc_input(x, degpart, in_W, in_b, ln_g, ln_b, c1_W):
    def body(x_ref, dp_ref, w_ref, b_ref, g_ref, lb_ref, w1_ref, y_ref):
        xb = x_ref[...]
        acc = jnp.zeros((_BLK, H), jnp.float32)
        for j in range(4):
            acc = acc + xb[:, j:j + 1] * w_ref[j:j + 1, :]
        h0 = jnp.tanh(_ln(acc + b_ref[...].reshape(1, H),
                          g_ref[...].reshape(1, H), lb_ref[...].reshape(1, H)))
        dis = _dis_of(dp_ref[...])
        y_ref[...] = dis * jnp.dot(h0, w1_ref[...],
                                   preferred_element_type=jnp.float32)

    return pl.pallas_call(
        body,
        grid=(_GRID,),
        in_specs=[
            pl.BlockSpec((_BLK, 4), lambda i: (i, 0)),
            pl.BlockSpec((NC, _BLK, WD), lambda i: (0, i, 0)),
            pl.BlockSpec((4, H), lambda i: (0, 0)),
            pl.BlockSpec((H,), lambda i: (0,)),
            pl.BlockSpec((H,), lambda i: (0,)),
            pl.BlockSpec((H,), lambda i: (0,)),
            pl.BlockSpec((H, H), lambda i: (0, 0)),
        ],
        out_specs=pl.BlockSpec((_BLK, H), lambda i: (i, 0)),
        out_shape=jax.ShapeDtypeStruct((N, H), jnp.float32),
    )(x, degpart, in_W, in_b, ln_g, ln_b, c1_W)


def _tc_conv_step(accpair, yprev, degpart, b_prev, W_next):
    def body(a_ref, y_ref, dp_ref, b_ref, w_ref, o_ref):
        dis = _dis_of(dp_ref[...])
        h = jax.nn.relu(dis * (a_ref[0] + a_ref[1] + y_ref[...]) +
                        b_ref[...].reshape(1, H))
        o_ref[...] = dis * jnp.dot(h, w_ref[...],
                                   preferred_element_type=jnp.float32)

    return pl.pallas_call(
        body,
        grid=(_GRID,),
        in_specs=[
            pl.BlockSpec((NC, _BLK, H), lambda i: (0, i, 0)),
            pl.BlockSpec((_BLK, H), lambda i: (i, 0)),
            pl.BlockSpec((NC, _BLK, WD), lambda i: (0, i, 0)),
            pl.BlockSpec((H,), lambda i: (0,)),
            pl.BlockSpec((H, H), lambda i: (0, 0)),
        ],
        out_specs=pl.BlockSpec((_BLK, H), lambda i: (i, 0)),
        out_shape=jax.ShapeDtypeStruct((N, H), jnp.float32),
    )(accpair, yprev, degpart, b_prev, W_next)


def _tc_assign(accpair, yprev, degpart, maskpart, b3,
               a1_W1, a1_b1, ln1_g, ln1_b, a1_W2, a1_b2, ln2_g, ln2_b):
    def body(a_ref, y_ref, dp_ref, mp_ref, b3_ref, w1_ref, b1_ref, g1_ref,
             lb1_ref, w2_ref, b2_ref, g2_ref, lb2_ref,
             h_ref, s_ref, sd_ref):
        dis = _dis_of(dp_ref[...])
        h3 = dis * (a_ref[0] + a_ref[1] + y_ref[...]) + b3_ref[...].reshape(1, H)
        h_ref[...] = h3
        t = jnp.tanh(_ln(jnp.dot(h3, w1_ref[...],
                                 preferred_element_type=jnp.float32) +
                         b1_ref[...].reshape(1, H),
                         g1_ref[...].reshape(1, H), lb1_ref[...].reshape(1, H)))
        s = jax.nn.relu(_ln(jnp.dot(t, w2_ref[...],
                                    preferred_element_type=jnp.float32) +
                            b2_ref[...].reshape(1, M0),
                            g2_ref[...].reshape(1, M0),
                            lb2_ref[...].reshape(1, M0)))
        s = s - jnp.max(s, axis=-1, keepdims=True)
        es = jnp.exp(s)
        S = es / jnp.sum(es, axis=-1, keepdims=True)
        s_ref[...] = S
        degD = jnp.sum(mp_ref[0], axis=-1, keepdims=True) + \
            jnp.sum(mp_ref[1], axis=-1, keepdims=True) + 1.0
        disD = lax.rsqrt(degD)
        sd_ref[:, 0:M0] = disD * S
        sd_ref[:, M0:M0 + 1] = disD
        sd_ref[:, M0 + 1:WD] = jnp.zeros((_BLK, WD - M0 - 1), jnp.float32)

    return pl.pallas_call(
        body,
        grid=(_GRID,),
        in_specs=[
            pl.BlockSpec((NC, _BLK, H), lambda i: (0, i, 0)),
            pl.BlockSpec((_BLK, H), lambda i: (i, 0)),
            pl.BlockSpec((NC, _BLK, WD), lambda i: (0, i, 0)),
            pl.BlockSpec((NC, _BLK, WD), lambda i: (0, i, 0)),
            pl.BlockSpec((H,), lambda i: (0,)),
            pl.BlockSpec((H, H), lambda i: (0, 0)),
            pl.BlockSpec((H,), lambda i: (0,)),
            pl.BlockSpec((H,), lambda i: (0,)),
            pl.BlockSpec((H,), lambda i: (0,)),
            pl.BlockSpec((H, M0), lambda i: (0, 0)),
            pl.BlockSpec((M0,), lambda i: (0,)),
            pl.BlockSpec((M0,), lambda i: (0,)),
            pl.BlockSpec((M0,), lambda i: (0,)),
        ],
        out_specs=[
            pl.BlockSpec((_BLK, H), lambda i: (i, 0)),
            pl.BlockSpec((_BLK, M0), lambda i: (i, 0)),
            pl.BlockSpec((_BLK, WD), lambda i: (i, 0)),
        ],
        out_shape=[
            jax.ShapeDtypeStruct((N, H), jnp.float32),
            jax.ShapeDtypeStruct((N, M0), jnp.float32),
            jax.ShapeDtypeStruct((N, WD), jnp.float32),
        ],
    )(accpair, yprev, degpart, maskpart, b3,
      a1_W1, a1_b1, ln1_g, ln1_b, a1_W2, a1_b2, ln2_g, ln2_b)


def _tc_tail(h3, S, sd80, tpart, p):
    def body(h_ref, s_ref, sd_ref, t_ref,
             c4w, c4b, c5w, c5b, c6w, c6b,
             a2w1, a2b1, a2g1, a2lb1, a2w2, a2b2, a2g2, a2lb2,
             ow1, ob1, og1, olb1, ow2, ob2, og2, olb2, ow3, ob3,
             gp_ref, ml1_ref, ol1_ref, ml2_ref, ol2_ref):
        Sm = s_ref[...]
        SD = sd_ref[:, 0:M0]
        disD = sd_ref[:, M0:M0 + 1]
        Tm = t_ref[0, :, 0:M0] + t_ref[1, :, 0:M0]
        rm = t_ref[0, :, M0:M0 + 1] + t_ref[1, :, M0:M0 + 1]

        cdim = (((0,), (0,)), ((), ()))
        out_adj = lax.dot_general(SD, Tm + SD, cdim,
                                  preferred_element_type=jnp.float32)
        out0 = lax.dot_general(Sm, h_ref[...], cdim,
                               preferred_element_type=jnp.float32)
        ss = lax.dot_general(Sm, Sm, cdim,
                             preferred_element_type=jnp.float32)
        d_flat = disD * (disD + rm)

        i0 = lax.broadcasted_iota(jnp.int32, (M0, M0), 0)
        i1 = lax.broadcasted_iota(jnp.int32, (M0, M0), 1)
        eye0 = jnp.where(i0 == i1, 1.0, 0.0)

        mincut_num = jnp.sum(out_adj * eye0)
        mincut_den = jnp.sum(d_flat * jnp.sum(Sm * Sm, axis=-1, keepdims=True))
        ml1 = -(mincut_num / mincut_den)

        ss_norm = jnp.sqrt(jnp.sum(ss * ss))
        ol1 = jnp.sqrt(jnp.sum((ss / ss_norm - eye0 / jnp.sqrt(
            jnp.float32(M0))) ** 2))

        out_adj = out_adj * (1.0 - eye0)
        dcol = jnp.sqrt(jnp.sum(out_adj, axis=-1, keepdims=True)) + 1e-15
        out_adj = out_adj / dcol / dcol.reshape(1, M0)

        def dense_gcn(xz, adj, W, b):
            adj = adj * (1.0 - eye0) + eye0
            dd = jnp.sum(adj, axis=-1, keepdims=True)
            ddis = jnp.where(dd > 0, lax.rsqrt(dd), 0.0)
            adjn = ddis * adj * ddis.reshape(1, M0)
            return jnp.dot(adjn, jnp.dot(xz, W,
                                         preferred_element_type=jnp.float32),
                           preferred_element_type=jnp.float32) + b.reshape(1, -1)

        z = jax.nn.relu(dense_gcn(out0, out_adj, c4w[...], c4b[...]))
        z = jax.nn.relu(dense_gcn(z, out_adj, c5w[...], c5b[...]))
        z = dense_gcn(z, out_adj, c6w[...], c6b[...])

        t2 = jnp.tanh(_ln(jnp.dot(z, a2w1[...],
                                  preferred_element_type=jnp.float32) +
                          a2b1[...].reshape(1, H),
                          a2g1[...].reshape(1, H), a2lb1[...].reshape(1, H)))
        s2 = jax.nn.relu(_ln(jnp.dot(t2, a2w2[...],
                                     preferred_element_type=jnp.float32) +
                             a2b2[...].reshape(1, M1),
                             a2g2[...].reshape(1, M1), a2lb2[...].reshape(1, M1)))
        s2 = s2 - jnp.max(s2, axis=-1, keepdims=True)
        e2 = jnp.exp(s2)
        S2 = e2 / jnp.sum(e2, axis=-1, keepdims=True)

        out2 = lax.dot_general(S2, z, cdim, preferred_element_type=jnp.float32)
        oa2 = lax.dot_general(S2, jnp.dot(out_adj, S2,
                                          preferred_element_type=jnp.float32),
                              cdim, preferred_element_type=jnp.float32)
        j0 = lax.broadcasted_iota(jnp.int32, (M1, M1), 0)
        j1 = lax.broadcasted_iota(jnp.int32, (M1, M1), 1)
        eye1 = jnp.where(j0 == j1, 1.0, 0.0)
        mn2 = jnp.sum(oa2 * eye1)
        dfl2 = jnp.sum(out_adj, axis=-1, keepdims=True)
        md2 = jnp.sum(dfl2 * jnp.sum(S2 * S2, axis=-1, keepdims=True))
        ml2 = -(mn2 / md2)
        ss2 = lax.dot_general(S2, S2, cdim, preferred_element_type=jnp.float32)
        ss2n = jnp.sqrt(jnp.sum(ss2 * ss2))
        ol2 = jnp.sqrt(jnp.sum((ss2 / ss2n - eye1 / jnp.sqrt(
            jnp.float32(M1))) ** 2))

        pooled = jnp.mean(out2, axis=0, keepdims=True)
        g1 = jnp.tanh(_ln(jnp.dot(pooled, ow1[...],
                                  preferred_element_type=jnp.float32) +
                          ob1[...].reshape(1, H),
                          og1[...].reshape(1, H), olb1[...].reshape(1, H)))
        g2 = jnp.tanh(_ln(jnp.dot(g1, ow2[...],
                                  preferred_element_type=jnp.float32) +
                          ob2[...].reshape(1, H),
                          og2[...].reshape(1, H), olb2[...].reshape(1, H)))
        gp = jnp.dot(g2, ow3[...], preferred_element_type=jnp.float32) + \
            ob3[...].reshape(1, 1)

        gp_ref[...] = gp
        ml1_ref[...] = jnp.reshape(ml1, (1, 1))
        ol1_ref[...] = jnp.reshape(ol1, (1, 1))
        ml2_ref[...] = jnp.reshape(ml2, (1, 1))
        ol2_ref[...] = jnp.reshape(ol2, (1, 1))

    args = (h3, S, sd80, tpart,
            p['c4_W'], p['c4_b'], p['c5_W'], p['c5_b'], p['c6_W'], p['c6_b'],
            p['a2_W1'], p['a2_b1'], p['a2_ln1_g'], p['a2_ln1_b'],
            p['a2_W2'], p['a2_b2'], p['a2_ln2_g'], p['a2_ln2_b'],
            p['o_W1'], p['o_b1'], p['o_ln1_g'], p['o_ln1_b'],
            p['o_W2'], p['o_b2'], p['o_ln2_g'], p['o_ln2_b'],
            p['o_W3'], p['o_b3'])
    out_shape = [jax.ShapeDtypeStruct((1, 1), jnp.float32)] * 5
    return pl.pallas_call(body, out_shape=out_shape)(*args)


def kernel(x, edge_index, batch, batch_size, params):
    p = params
    edge = edge_index.astype(jnp.int32)
    eidx = lax.iota(jnp.int32, E)

    pwin, degpart = _sc_edge_prep(edge, eidx)
    y1p = _tc_input(x, degpart, p['in_W'], p['in_b'], p['in_ln_g'],
                    p['in_ln_b'], p['c1_W'])
    tgt, maskpart, acc1 = _sc_mask_agg(pwin, edge, eidx, y1p)
    y2p = _tc_conv_step(acc1, y1p, degpart, p['c1_b'], p['c2_W'])
    acc2 = _sc_conv_agg(edge, y2p)
    y3p = _tc_conv_step(acc2, y2p, degpart, p['c2_b'], p['c3_W'])
    acc3 = _sc_conv_agg(edge, y3p)
    h3, S, sd80 = _tc_assign(acc3, y3p, degpart, maskpart, p['c3_b'],
                             p['a1_W1'], p['a1_b1'], p['a1_ln1_g'],
                             p['a1_ln1_b'], p['a1_W2'], p['a1_b2'],
                             p['a1_ln2_g'], p['a1_ln2_b'])
    tpart = _sc_pool_agg(edge, tgt, sd80)
    gp, ml1, ol1, ml2, ol2 = _tc_tail(h3, S, sd80, tpart, p)
    return (gp, ml1.reshape(()), ol1.reshape(()), ml2.reshape(()),
            ol2.reshape(()))
